# baseline (speedup 1.0000x reference)
import functools

import jax
import jax.numpy as jnp
from jax.experimental import pallas as pl
from jax.experimental.pallas import tpu as pltpu

_TILE = 128
_PROJ_CHUNK = 256


def _row_sumsq(x):
    s = x * x
    acc = s
    for k in range(1, 16):
        acc = acc + pltpu.roll(s, 128 - 8 * k, 1)
    for j in (4, 2, 1):
        acc = acc + pltpu.roll(acc, 128 - j, 1)
    return acc[:, 0:1]


def _vq_body(z_ref, w_in_ref, b_in_ref, cb_ref, hi_ref, mid_ref, lo_ref,
             e2_ref, w_out_ref, b_out_ref,
             codes_ref, out_ref, commit_ref, *, n_books, n_k):
    @pl.when(pl.program_id(0) == 0)
    def _init():
        commit_ref[...] = jnp.zeros((1, 1), jnp.float32)

    zz = z_ref[...]
    ww = w_in_ref[...]
    r = None
    for c0 in range(0, zz.shape[1], _PROJ_CHUNK):
        part = jnp.dot(zz[:, c0:c0 + _PROJ_CHUNK], ww[c0:c0 + _PROJ_CHUNK, :])
        r = part if r is None else r + part
    r = r + b_in_ref[...]

    qsum = jnp.zeros_like(r)
    csum = jnp.float32(0.0)
    iota = jax.lax.broadcasted_iota(jnp.int32, (r.shape[0], n_k), 1)
    for i in range(n_books):
        s2 = jnp.dot(r, cb_ref[i])
        rsq = _row_sumsq(r)
        dist = (rsq + e2_ref[i]) + s2
        m = jnp.min(dist, axis=1, keepdims=True)
        codes = jnp.min(jnp.where(dist == m, iota, n_k), axis=1)
        codes_ref[i, :] = codes.astype(jnp.int32)
        oh = (iota == codes[:, None]).astype(jnp.bfloat16)
        qh = jnp.dot(oh, hi_ref[i], preferred_element_type=jnp.float32)
        qm = jnp.dot(oh, mid_ref[i], preferred_element_type=jnp.float32)
        ql = jnp.dot(oh, lo_ref[i], preferred_element_type=jnp.float32)
        q = (qh + qm) + ql
        dq = q - r
        q_st = r + dq
        qsum = qsum + q_st
        r = r - q_st
        csum = csum + jnp.sum(dq * dq)
    commit_ref[...] = commit_ref[...] + csum
    out_ref[...] = jnp.dot(qsum, w_out_ref[...]) + b_out_ref[...]


def kernel(z, W_in, b_in, W_out, b_out, codebooks):
    B, C, T = z.shape
    NB, K, D = codebooks.shape
    BT = B * T
    z_t = z.transpose(0, 2, 1).reshape(BT, C)
    cb_t = -2.0 * codebooks.transpose(0, 2, 1)
    cb_hi = codebooks.astype(jnp.bfloat16)
    res1 = codebooks - cb_hi.astype(jnp.float32)
    cb_mid = res1.astype(jnp.bfloat16)
    cb_lo = (res1 - cb_mid.astype(jnp.float32)).astype(jnp.bfloat16)
    e2 = jnp.sum(codebooks ** 2, axis=-1)[:, None, :]
    grid = BT // _TILE
    codes_flat, out_t, commit = pl.pallas_call(
        functools.partial(_vq_body, n_books=NB, n_k=K),
        grid=(grid,),
        in_specs=[
            pl.BlockSpec((_TILE, C), lambda i: (i, 0)),
            pl.BlockSpec((C, D), lambda i: (0, 0)),
            pl.BlockSpec((1, D), lambda i: (0, 0)),
            pl.BlockSpec((NB, D, K), lambda i: (0, 0, 0)),
            pl.BlockSpec((NB, K, D), lambda i: (0, 0, 0)),
            pl.BlockSpec((NB, K, D), lambda i: (0, 0, 0)),
            pl.BlockSpec((NB, K, D), lambda i: (0, 0, 0)),
            pl.BlockSpec((NB, 1, K), lambda i: (0, 0, 0)),
            pl.BlockSpec((D, C), lambda i: (0, 0)),
            pl.BlockSpec((1, C), lambda i: (0, 0)),
        ],
        out_specs=[
            pl.BlockSpec((NB, _TILE), lambda i: (0, i)),
            pl.BlockSpec((_TILE, C), lambda i: (i, 0)),
            pl.BlockSpec((1, 1), lambda i: (0, 0)),
        ],
        out_shape=[
            jax.ShapeDtypeStruct((NB, BT), jnp.int32),
            jax.ShapeDtypeStruct((BT, C), jnp.float32),
            jax.ShapeDtypeStruct((1, 1), jnp.float32),
        ],
    )(z_t, W_in.T, b_in.reshape(1, D), cb_t, cb_hi, cb_mid, cb_lo, e2,
      W_out.T, b_out.reshape(1, C))
    codes_all = codes_flat.reshape(NB, B, T).transpose(1, 0, 2)
    out = out_t.reshape(B, T, C).transpose(0, 2, 1)
    total_commit = commit[0, 0] / jnp.float32(NB * BT * D)
    return codes_all, out, total_commit


# speedup vs baseline: 1.2266x; 1.2266x over previous
import functools

import jax
import jax.numpy as jnp
from jax.experimental import pallas as pl
from jax.experimental.pallas import tpu as pltpu

_TILE = 256
_PROJ_CHUNK = 256


def _row_sumsq(x):
    s = x * x
    acc = s
    for k in range(1, 16):
        acc = acc + pltpu.roll(s, 128 - 8 * k, 1)
    for j in (4, 2, 1):
        acc = acc + pltpu.roll(acc, 128 - j, 1)
    return acc[:, 0:1]


def _vq_body(z_ref, w_in_ref, b_in_ref, cb_ref, hi_ref, mid_ref, lo_ref,
             e2_ref, w_out_ref, b_out_ref,
             codes_ref, out_ref, commit_ref, *, n_books, n_k):
    @pl.when(pl.program_id(0) == 0)
    def _init():
        commit_ref[...] = jnp.zeros((1, 1), jnp.float32)

    zz = z_ref[...]
    ww = w_in_ref[...]
    r = None
    for c0 in range(0, zz.shape[1], _PROJ_CHUNK):
        part = jnp.dot(zz[:, c0:c0 + _PROJ_CHUNK], ww[c0:c0 + _PROJ_CHUNK, :])
        r = part if r is None else r + part
    r = r + b_in_ref[...]

    qsum = jnp.zeros_like(r)
    csum = jnp.float32(0.0)
    iota = jax.lax.broadcasted_iota(jnp.int32, (r.shape[0], n_k), 1)
    for i in range(n_books):
        s2 = jnp.dot(r, cb_ref[i])
        rsq = _row_sumsq(r)
        dist = (rsq + e2_ref[i]) + s2
        m = jnp.min(dist, axis=1, keepdims=True)
        codes = jnp.min(jnp.where(dist == m, iota, n_k), axis=1)
        codes_ref[i, :] = codes.astype(jnp.int32)
        oh = (iota == codes[:, None]).astype(jnp.bfloat16)
        qh = jnp.dot(oh, hi_ref[i], preferred_element_type=jnp.float32)
        qm = jnp.dot(oh, mid_ref[i], preferred_element_type=jnp.float32)
        ql = jnp.dot(oh, lo_ref[i], preferred_element_type=jnp.float32)
        q = (qh + qm) + ql
        dq = q - r
        q_st = r + dq
        qsum = qsum + q_st
        r = r - q_st
        csum = csum + jnp.sum(dq * dq)
    commit_ref[...] = commit_ref[...] + csum
    out_ref[...] = jnp.dot(qsum, w_out_ref[...]) + b_out_ref[...]


def kernel(z, W_in, b_in, W_out, b_out, codebooks):
    B, C, T = z.shape
    NB, K, D = codebooks.shape
    BT = B * T
    z_t = z.transpose(0, 2, 1).reshape(BT, C)
    cb_t = -2.0 * codebooks.transpose(0, 2, 1)
    cb_hi = codebooks.astype(jnp.bfloat16)
    res1 = codebooks - cb_hi.astype(jnp.float32)
    cb_mid = res1.astype(jnp.bfloat16)
    cb_lo = (res1 - cb_mid.astype(jnp.float32)).astype(jnp.bfloat16)
    e2 = jnp.sum(codebooks ** 2, axis=-1)[:, None, :]
    grid = BT // _TILE
    codes_flat, out_t, commit = pl.pallas_call(
        functools.partial(_vq_body, n_books=NB, n_k=K),
        grid=(grid,),
        in_specs=[
            pl.BlockSpec((_TILE, C), lambda i: (i, 0)),
            pl.BlockSpec((C, D), lambda i: (0, 0)),
            pl.BlockSpec((1, D), lambda i: (0, 0)),
            pl.BlockSpec((NB, D, K), lambda i: (0, 0, 0)),
            pl.BlockSpec((NB, K, D), lambda i: (0, 0, 0)),
            pl.BlockSpec((NB, K, D), lambda i: (0, 0, 0)),
            pl.BlockSpec((NB, K, D), lambda i: (0, 0, 0)),
            pl.BlockSpec((NB, 1, K), lambda i: (0, 0, 0)),
            pl.BlockSpec((D, C), lambda i: (0, 0)),
            pl.BlockSpec((1, C), lambda i: (0, 0)),
        ],
        out_specs=[
            pl.BlockSpec((NB, _TILE), lambda i: (0, i)),
            pl.BlockSpec((_TILE, C), lambda i: (i, 0)),
            pl.BlockSpec((1, 1), lambda i: (0, 0)),
        ],
        out_shape=[
            jax.ShapeDtypeStruct((NB, BT), jnp.int32),
            jax.ShapeDtypeStruct((BT, C), jnp.float32),
            jax.ShapeDtypeStruct((1, 1), jnp.float32),
        ],
    )(z_t, W_in.T, b_in.reshape(1, D), cb_t, cb_hi, cb_mid, cb_lo, e2,
      W_out.T, b_out.reshape(1, C))
    codes_all = codes_flat.reshape(NB, B, T).transpose(1, 0, 2)
    out = out_t.reshape(B, T, C).transpose(0, 2, 1)
    total_commit = commit[0, 0] / jnp.float32(NB * BT * D)
    return codes_all, out, total_commit
